# Initial kernel scaffold; baseline (speedup 1.0000x reference)
import jax
import jax.numpy as jnp
from jax.experimental import pallas as pl


def kernel(token_ids, edge_src, emb_table, W_weight, W_bias, fc_weight, fc_bias):
    raise NotImplementedError("write your pallas kernel here")


# speedup vs baseline: 1.7761x; 1.7761x over previous
import functools

import jax
import jax.numpy as jnp
from jax import lax
from jax.experimental import pallas as pl
from jax.experimental.pallas import tpu as pltpu
from jax.experimental.pallas import tpu_sc as plsc

N_NODES = 10000
DEG = 32
D = 128
N_CLASSES = 104
N_EDGES = N_NODES * DEG

NW = 32
NB = 316
P = NW * NB
CH = 4
NCHUNK = NB // CH
EPC = CH * DEG

_mesh = plsc.VectorSubcoreMesh(core_axis_name="c", subcore_axis_name="s")


@functools.partial(
    pl.kernel,
    mesh=_mesh,
    out_type=[
        jax.ShapeDtypeStruct((P, D), jnp.float32),
        jax.ShapeDtypeStruct((P, D), jnp.float32),
    ],
    scratch_types=[
        pltpu.VMEM((NCHUNK, EPC), jnp.int32),
        pltpu.VMEM((NCHUNK, EPC), jnp.int32),
        pltpu.VMEM((EPC, D), jnp.float32),
        pltpu.VMEM((CH, D), jnp.float32),
        pltpu.VMEM((CH, D), jnp.float32),
        pltpu.SemaphoreType.DMA,
    ],
)
def _sc_gather_reduce(edge_hbm, tok_hbm, emb_hbm, tot_hbm, last_hbm,
                      edge_v, comb_v, rows_v, tot_v, last_v, sem):
    wid = lax.axis_index("s") * 2 + lax.axis_index("c")
    pltpu.sync_copy(edge_hbm.at[wid], edge_v)

    def gather_tokens(c, carry):
        pltpu.async_copy(tok_hbm.at[edge_v.at[c]], comb_v.at[c], sem).wait()
        return carry

    lax.fori_loop(0, NCHUNK, gather_tokens, 0)

    base_node = wid * NB

    def body(c, carry):
        pltpu.async_copy(emb_hbm.at[comb_v.at[c]], rows_v, sem).wait()
        for n in range(CH):
            for d in range(D // 16):
                sl = pl.ds(d * 16, 16)
                acc = rows_v[n * DEG, sl]
                for r in range(1, DEG):
                    acc = acc + rows_v[n * DEG + r, sl]
                tot_v[n, sl] = acc
                last_v[n, sl] = rows_v[n * DEG + DEG - 1, sl]
        row0 = base_node + c * CH
        pltpu.sync_copy(tot_v, tot_hbm.at[pl.ds(row0, CH)])
        pltpu.sync_copy(last_v, last_hbm.at[pl.ds(row0, CH)])
        return carry

    lax.fori_loop(0, NCHUNK, body, 0)


BLK = P // 8


def _tc_head_body(tot_ref, last_ref, w_ref, b_ref, fc_ref, fcb_ref, out_ref):
    tot = tot_ref[...]
    last = last_ref[...]
    pre = lax.dot_general(tot - last, w_ref[...], (((1,), (1,)), ((), ())),
                          preferred_element_type=jnp.float32)
    h = last + jnp.maximum(pre + b_ref[...], 0.0)
    out = lax.dot_general(h, fc_ref[...], (((1,), (1,)), ((), ())),
                          preferred_element_type=jnp.float32)
    out_ref[...] = out + fcb_ref[...]


def _tc_head(tot, last, W_weight, W_bias, fc_weight, fc_bias):
    return pl.pallas_call(
        _tc_head_body,
        grid=(P // BLK,),
        in_specs=[
            pl.BlockSpec((BLK, D), lambda i: (i, 0)),
            pl.BlockSpec((BLK, D), lambda i: (i, 0)),
            pl.BlockSpec((D, D), lambda i: (0, 0)),
            pl.BlockSpec((1, D), lambda i: (0, 0)),
            pl.BlockSpec((N_CLASSES, D), lambda i: (0, 0)),
            pl.BlockSpec((1, N_CLASSES), lambda i: (0, 0)),
        ],
        out_specs=pl.BlockSpec((BLK, N_CLASSES), lambda i: (i, 0)),
        out_shape=jax.ShapeDtypeStruct((P, N_CLASSES), jnp.float32),
    )(tot, last, W_weight, W_bias.reshape(1, D),
      fc_weight, fc_bias.reshape(1, N_CLASSES))


def kernel(token_ids, edge_src, emb_table, W_weight, W_bias, fc_weight, fc_bias):
    tok = token_ids.astype(jnp.int32)
    es = edge_src.astype(jnp.int32)
    es_p = jnp.pad(es, (0, P * DEG - N_EDGES)).reshape(NW, NCHUNK, EPC)
    tot, last = _sc_gather_reduce(es_p, tok, emb_table)
    out = _tc_head(tot, last, W_weight, W_bias, fc_weight, fc_bias)
    return out[:N_NODES]


# speedup vs baseline: 1.8219x; 1.0258x over previous
import functools

import jax
import jax.numpy as jnp
from jax import lax
from jax.experimental import pallas as pl
from jax.experimental.pallas import tpu as pltpu
from jax.experimental.pallas import tpu_sc as plsc

N_NODES = 10000
DEG = 32
D = 128
N_CLASSES = 104
N_EDGES = N_NODES * DEG

NW = 32
NB = 320
P = NW * NB
CH = 4
NCHUNK = NB // CH
EPC = CH * DEG

_mesh = plsc.VectorSubcoreMesh(core_axis_name="c", subcore_axis_name="s")


@functools.partial(
    pl.kernel,
    mesh=_mesh,
    out_type=jax.ShapeDtypeStruct((P, 2, D), jnp.float32),
    scratch_types=[
        pltpu.VMEM((NCHUNK, EPC), jnp.int32),
        pltpu.VMEM((NCHUNK, EPC), jnp.int32),
        pltpu.VMEM((EPC, D), jnp.float32),
        pltpu.VMEM((EPC, D), jnp.float32),
        pltpu.VMEM((CH, 2, D), jnp.float32),
        pltpu.VMEM((CH, 2, D), jnp.float32),
        pltpu.SemaphoreType.DMA,
        pltpu.SemaphoreType.DMA,
        pltpu.SemaphoreType.DMA,
        pltpu.SemaphoreType.DMA,
        pltpu.SemaphoreType.DMA,
    ],
)
def _sc_gather_reduce(edge_hbm, tok_hbm, emb_hbm, tl_hbm,
                      edge_v, comb_v, rows_a, rows_b, tl_a, tl_b,
                      sem_t, sem_ra, sem_rb, sem_wa, sem_wb):
    wid = lax.axis_index("s") * 2 + lax.axis_index("c")
    pltpu.sync_copy(edge_hbm.at[wid], edge_v)
    base_node = wid * NB

    def fire_tok(c):
        pltpu.async_copy(tok_hbm.at[edge_v.at[c]], comb_v.at[c], sem_t)

    def wait_tok():
        pltpu.make_async_copy(tok_hbm.at[edge_v.at[0]], comb_v.at[0],
                              sem_t).wait()

    def fire_row(c, buf, sem):
        pltpu.async_copy(emb_hbm.at[comb_v.at[c]], buf, sem)

    def wait_row(buf, sem):
        pltpu.make_async_copy(emb_hbm.at[comb_v.at[0]], buf, sem).wait()

    def wait_write(buf, sem):
        pltpu.make_async_copy(buf, tl_hbm.at[pl.ds(0, CH)], sem).wait()

    def reduce_chunk(rows, tl):
        for n in range(CH):
            for d in range(D // 16):
                sl = pl.ds(d * 16, 16)
                acc = rows[n * DEG, sl]
                for r in range(1, DEG):
                    acc = acc + rows[n * DEG + r, sl]
                tl[n, 0, sl] = acc
                tl[n, 1, sl] = rows[n * DEG + DEG - 1, sl]

    fire_tok(0)
    wait_tok()
    fire_row(0, rows_a, sem_ra)
    fire_tok(1)

    def body(t, carry):
        e = 2 * t
        o = e + 1
        wait_tok()
        fire_row(o, rows_b, sem_rb)

        @pl.when(e + 2 < NCHUNK)
        def _():
            fire_tok(e + 2)

        wait_row(rows_a, sem_ra)

        @pl.when(t > 0)
        def _():
            wait_write(tl_a, sem_wa)

        reduce_chunk(rows_a, tl_a)
        pltpu.async_copy(tl_a, tl_hbm.at[pl.ds(base_node + e * CH, CH)],
                         sem_wa)

        @pl.when(e + 2 < NCHUNK)
        def _():
            wait_tok()
            fire_row(e + 2, rows_a, sem_ra)

        @pl.when(e + 3 < NCHUNK)
        def _():
            fire_tok(e + 3)

        wait_row(rows_b, sem_rb)

        @pl.when(t > 0)
        def _():
            wait_write(tl_b, sem_wb)

        reduce_chunk(rows_b, tl_b)
        pltpu.async_copy(tl_b, tl_hbm.at[pl.ds(base_node + o * CH, CH)],
                         sem_wb)
        return carry

    lax.fori_loop(0, NCHUNK // 2, body, 0)
    wait_write(tl_a, sem_wa)
    wait_write(tl_b, sem_wb)


BLK = P // 8


def _tc_head_body(tl_ref, w_ref, b_ref, fc_ref, fcb_ref, out_ref):
    tot = tl_ref[:, 0, :]
    last = tl_ref[:, 1, :]
    pre = lax.dot_general(tot - last, w_ref[...], (((1,), (1,)), ((), ())),
                          preferred_element_type=jnp.float32)
    h = last + jnp.maximum(pre + b_ref[...], 0.0)
    out = lax.dot_general(h, fc_ref[...], (((1,), (1,)), ((), ())),
                          preferred_element_type=jnp.float32)
    out_ref[...] = out + fcb_ref[...]


def _tc_head(tl, W_weight, W_bias, fc_weight, fc_bias):
    return pl.pallas_call(
        _tc_head_body,
        grid=(P // BLK,),
        in_specs=[
            pl.BlockSpec((BLK, 2, D), lambda i: (i, 0, 0)),
            pl.BlockSpec((D, D), lambda i: (0, 0)),
            pl.BlockSpec((1, D), lambda i: (0, 0)),
            pl.BlockSpec((N_CLASSES, D), lambda i: (0, 0)),
            pl.BlockSpec((1, N_CLASSES), lambda i: (0, 0)),
        ],
        out_specs=pl.BlockSpec((BLK, N_CLASSES), lambda i: (i, 0)),
        out_shape=jax.ShapeDtypeStruct((P, N_CLASSES), jnp.float32),
    )(tl, W_weight, W_bias.reshape(1, D),
      fc_weight, fc_bias.reshape(1, N_CLASSES))


def kernel(token_ids, edge_src, emb_table, W_weight, W_bias, fc_weight, fc_bias):
    tok = token_ids.astype(jnp.int32)
    es = edge_src.astype(jnp.int32)
    es_p = jnp.pad(es, (0, P * DEG - N_EDGES)).reshape(NW, NCHUNK, EPC)
    tl = _sc_gather_reduce(es_p, tok, emb_table)
    out = _tc_head(tl, W_weight, W_bias, fc_weight, fc_bias)
    return out[:N_NODES]
